# Initial kernel scaffold; baseline (speedup 1.0000x reference)
import jax
import jax.numpy as jnp
from jax.experimental import pallas as pl


def kernel(indices, embeddings):
    raise NotImplementedError("write your pallas kernel here")


# speedup vs baseline: 5.0495x; 5.0495x over previous
import functools

import jax
import jax.numpy as jnp
from jax import lax
from jax.experimental import pallas as pl
from jax.experimental.pallas import tpu as pltpu
from jax.experimental.pallas import tpu_sc as plsc

_VOCAB = 100000
_DIM = 32
_B_TOT = 4096 * 200

_NC = 2
_NS = 16
_NW = _NC * _NS
_B_PER_W = _B_TOT // _NW
_CHUNK = 1024
_NCHUNK = _B_PER_W // _CHUNK

_mesh = plsc.VectorSubcoreMesh(core_axis_name="c", subcore_axis_name="s")


@functools.partial(
    pl.kernel,
    mesh=_mesh,
    out_type=jax.ShapeDtypeStruct((_B_TOT, _DIM), jnp.float32),
    scratch_types=[
        pltpu.VMEM((_CHUNK,), jnp.int32),
        pltpu.VMEM((_CHUNK, _DIM), jnp.float32),
        pltpu.SemaphoreType.DMA,
    ],
    compiler_params=pltpu.CompilerParams(use_tc_tiling_on_sc=False),
)
def _gather_all(idx_hbm, table_hbm, out_hbm, idx_v, rows_v, sem):
    wid = lax.axis_index("s") * _NC + lax.axis_index("c")
    base = wid * _B_PER_W

    def body(g, carry):
        off = pl.multiple_of(base + g * _CHUNK, _CHUNK)
        pltpu.sync_copy(idx_hbm.at[pl.ds(off, _CHUNK)], idx_v)
        pltpu.async_copy(table_hbm.at[idx_v], rows_v, sem).wait()
        pltpu.sync_copy(rows_v, out_hbm.at[pl.ds(off, _CHUNK)])
        return carry

    lax.fori_loop(0, _NCHUNK, body, 0)


def kernel(indices, embeddings):
    idx = indices.astype(jnp.int32).reshape(-1)
    out = _gather_all(idx, embeddings)
    return out.reshape(indices.shape + (embeddings.shape[1],))


# speedup vs baseline: 5.2861x; 1.0469x over previous
import functools

import jax
import jax.numpy as jnp
from jax import lax
from jax.experimental import pallas as pl
from jax.experimental.pallas import tpu as pltpu
from jax.experimental.pallas import tpu_sc as plsc

_VOCAB = 100000
_DIM = 32
_B_TOT = 4096 * 200

_NC = 2
_NS = 16
_NW = _NC * _NS
_B_PER_W = _B_TOT // _NW
_CHUNK = 1600
_NCHUNK = _B_PER_W // _CHUNK

_mesh = plsc.VectorSubcoreMesh(core_axis_name="c", subcore_axis_name="s")


@functools.partial(
    pl.kernel,
    mesh=_mesh,
    out_type=jax.ShapeDtypeStruct((_B_TOT, _DIM), jnp.float32),
    scratch_types=[
        pltpu.VMEM((2, _CHUNK), jnp.int32),
        pltpu.VMEM((2, _CHUNK, _DIM), jnp.float32),
        pltpu.SemaphoreType.DMA((2,)),
        pltpu.SemaphoreType.DMA((2,)),
        pltpu.SemaphoreType.DMA((2,)),
    ],
    compiler_params=pltpu.CompilerParams(use_tc_tiling_on_sc=False),
)
def _gather_all(idx_hbm, table_hbm, out_hbm, idx_v, rows_v, sem_i, sem_g, sem_o):
    wid = lax.axis_index("s") * _NC + lax.axis_index("c")
    base = wid * _B_PER_W

    def off(g):
        return pl.multiple_of(base + g * _CHUNK, 8)

    def idx_copy(g):
        b = g % 2
        return pltpu.make_async_copy(
            idx_hbm.at[pl.ds(off(g), _CHUNK)], idx_v.at[b], sem_i.at[b])

    def gather(g):
        b = g % 2
        return pltpu.make_async_copy(
            table_hbm.at[idx_v.at[b]], rows_v.at[b], sem_g.at[b])

    def scatter(g):
        b = g % 2
        return pltpu.make_async_copy(
            rows_v.at[b], out_hbm.at[pl.ds(off(g), _CHUNK)], sem_o.at[b])

    idx_copy(0).start()
    idx_copy(1).start()
    idx_copy(0).wait()
    gather(0).start()
    for g in range(_NCHUNK):
        if g + 1 < _NCHUNK:
            idx_copy(g + 1).wait()
            if g >= 1:
                scatter(g - 1).wait()
            gather(g + 1).start()
        gather(g).wait()
        if g + 2 < _NCHUNK:
            idx_copy(g + 2).start()
        scatter(g).start()
    scatter(_NCHUNK - 2).wait()
    scatter(_NCHUNK - 1).wait()


def kernel(indices, embeddings):
    idx = indices.astype(jnp.int32).reshape(-1)
    out = _gather_all(idx, embeddings)
    return out.reshape(indices.shape + (embeddings.shape[1],))


# speedup vs baseline: 5.2886x; 1.0005x over previous
import functools

import jax
import jax.numpy as jnp
from jax import lax
from jax.experimental import pallas as pl
from jax.experimental.pallas import tpu as pltpu
from jax.experimental.pallas import tpu_sc as plsc

_VOCAB = 100000
_DIM = 32
_B_TOT = 4096 * 200

_NC = 2
_NS = 16
_NW = _NC * _NS
_B_PER_W = _B_TOT // _NW
_NBUF = 4
_CHUNK = 800
_NCHUNK = _B_PER_W // _CHUNK
_LAG = _NBUF - 1

_mesh = plsc.VectorSubcoreMesh(core_axis_name="c", subcore_axis_name="s")


@functools.partial(
    pl.kernel,
    mesh=_mesh,
    out_type=jax.ShapeDtypeStruct((_B_TOT, _DIM), jnp.float32),
    scratch_types=[
        pltpu.VMEM((_NBUF, _CHUNK), jnp.int32),
        pltpu.VMEM((_NBUF, _CHUNK, _DIM), jnp.float32),
        pltpu.SemaphoreType.DMA((_NBUF,)),
        pltpu.SemaphoreType.DMA((_NBUF,)),
        pltpu.SemaphoreType.DMA((_NBUF,)),
    ],
    compiler_params=pltpu.CompilerParams(use_tc_tiling_on_sc=False),
)
def _gather_all(idx_hbm, table_hbm, out_hbm, idx_v, rows_v, sem_i, sem_g, sem_o):
    wid = lax.axis_index("s") * _NC + lax.axis_index("c")
    base = wid * _B_PER_W

    def off(g):
        return pl.multiple_of(base + g * _CHUNK, 8)

    def idx_copy(g):
        b = g % _NBUF
        return pltpu.make_async_copy(
            idx_hbm.at[pl.ds(off(g), _CHUNK)], idx_v.at[b], sem_i.at[b])

    def gather(g):
        b = g % _NBUF
        return pltpu.make_async_copy(
            table_hbm.at[idx_v.at[b]], rows_v.at[b], sem_g.at[b])

    def scatter(g):
        b = g % _NBUF
        return pltpu.make_async_copy(
            rows_v.at[b], out_hbm.at[pl.ds(off(g), _CHUNK)], sem_o.at[b])

    for g in range(_NBUF):
        idx_copy(g).start()
    for g in range(_NCHUNK + _LAG):
        if g < _NCHUNK:
            idx_copy(g).wait()
            if g >= _NBUF:
                scatter(g - _NBUF).wait()
            gather(g).start()
        d = g - _LAG
        if d >= 0:
            gather(d).wait()
            if d + _NBUF < _NCHUNK:
                idx_copy(d + _NBUF).start()
            scatter(d).start()
    for d in range(_NCHUNK - _NBUF, _NCHUNK):
        scatter(d).wait()


def kernel(indices, embeddings):
    idx = indices.astype(jnp.int32).reshape(-1)
    out = _gather_all(idx, embeddings)
    return out.reshape(indices.shape + (embeddings.shape[1],))
